# baseline (speedup 1.0000x reference)
import functools

import jax
import jax.numpy as jnp
from jax import lax
from jax.experimental import pallas as pl
from jax.experimental.pallas import tpu as pltpu
from jax.experimental.pallas import tpu_sc as plsc

VOCAB = 49408
MAXPOS = 8192
DIM = 1024
NSPECIAL = 16
LROWS = MAXPOS + NSPECIAL

NC = 2
NS = 16
LANES = 16
NW = NC * NS
RW = MAXPOS // NW
CHUNK = 32
NCHUNK = RW // CHUNK
VPR = DIM // LANES


def _sc_body(ids_hbm, tok_hbm, pos_hbm, spec_hbm, out_hbm,
             idx_v, tok_v, pos_v, spec_v, gsem, psem):
    wid = lax.axis_index("s") * NC + lax.axis_index("c")
    base = wid * RW

    @pl.when(wid == NW - 1)
    def _():
        pltpu.sync_copy(spec_hbm, spec_v)
        pltpu.sync_copy(spec_v, out_hbm.at[pl.ds(1, NSPECIAL)])

    for ch in range(NCHUNK):
        row0 = base + ch * CHUNK
        pltpu.sync_copy(ids_hbm.at[pl.ds(NSPECIAL + ch * CHUNK + base, CHUNK)],
                        idx_v)
        g = pltpu.async_copy(tok_hbm.at[idx_v], tok_v, gsem)
        p = pltpu.async_copy(pos_hbm.at[pl.ds(row0, CHUNK)], pos_v, psem)
        g.wait()
        p.wait()

        def add_row(r, carry):
            for k in range(VPR):
                sl = pl.ds(k * LANES, LANES)
                plsc.addupdate(tok_v.at[r, sl], pos_v[r, sl])
            return carry
        lax.fori_loop(0, CHUNK, add_row, 0)

        if ch == 0:
            @pl.when(wid == 0)
            def _():
                pltpu.sync_copy(tok_v.at[pl.ds(0, 1)], out_hbm.at[pl.ds(0, 1)])
                pltpu.sync_copy(tok_v.at[pl.ds(1, CHUNK - 1)],
                                out_hbm.at[pl.ds(NSPECIAL + 1, CHUNK - 1)])

            @pl.when(wid != 0)
            def _():
                pltpu.sync_copy(tok_v, out_hbm.at[pl.ds(row0 + NSPECIAL, CHUNK)])
        else:
            pltpu.sync_copy(tok_v, out_hbm.at[pl.ds(row0 + NSPECIAL, CHUNK)])


_sc_kernel = functools.partial(
    pl.kernel,
    out_type=jax.ShapeDtypeStruct((LROWS, DIM), jnp.float32),
    mesh=plsc.VectorSubcoreMesh(core_axis_name="c", subcore_axis_name="s"),
    scratch_types=[
        pltpu.VMEM((CHUNK,), jnp.int32),
        pltpu.VMEM((CHUNK, DIM), jnp.float32),
        pltpu.VMEM((CHUNK, DIM), jnp.float32),
        pltpu.VMEM((NSPECIAL, DIM), jnp.float32),
        pltpu.SemaphoreType.DMA,
        pltpu.SemaphoreType.DMA,
    ],
    compiler_params=pltpu.CompilerParams(use_tc_tiling_on_sc=False),
)(_sc_body)


def kernel(input_ids, token_table, pos_table, special_token_embedding):
    ids = input_ids.reshape(LROWS)
    spec = special_token_embedding.reshape(NSPECIAL, DIM)
    out = _sc_kernel(ids, token_table, pos_table, spec)
    return out.reshape(1, LROWS, DIM)


# speedup vs baseline: 2.9434x; 2.9434x over previous
import functools

import jax
import jax.numpy as jnp
from jax import lax
from jax.experimental import pallas as pl
from jax.experimental.pallas import tpu as pltpu
from jax.experimental.pallas import tpu_sc as plsc

VOCAB = 49408
MAXPOS = 8192
DIM = 1024
NSPECIAL = 16
LROWS = MAXPOS + NSPECIAL

NC = 2
NS = 16
LANES = 16
NW = NC * NS
RW = MAXPOS // NW
CHUNK = 32
NCHUNK = RW // CHUNK
VPR = DIM // LANES
HEAD = NSPECIAL + CHUNK


def _sc_body(ids_hbm, tok_hbm, pos_hbm, spec_hbm, out_hbm,
             idx_v, tok_v, pos_v, stage_v, gsem, psem):
    wid = lax.axis_index("s") * NC + lax.axis_index("c")
    base = wid * RW

    @pl.when(wid == 0)
    def _():
        pltpu.sync_copy(spec_hbm, stage_v.at[pl.ds(CHUNK, NSPECIAL)])

        def cp_row(r, carry):
            for k in range(VPR):
                sl = pl.ds(k * LANES, LANES)
                stage_v[1 + r, sl] = stage_v[CHUNK + r, sl]
            return carry
        lax.fori_loop(0, NSPECIAL, cp_row, 0)

    for ch in range(NCHUNK):
        row0 = base + ch * CHUNK
        pltpu.sync_copy(ids_hbm.at[pl.ds(NSPECIAL + ch * CHUNK + base, CHUNK)],
                        idx_v)
        g = pltpu.async_copy(tok_hbm.at[idx_v], tok_v, gsem)
        p = pltpu.async_copy(pos_hbm.at[pl.ds(row0, CHUNK)], pos_v, psem)
        g.wait()
        p.wait()

        def add_row(r, carry):
            for k in range(VPR):
                sl = pl.ds(k * LANES, LANES)
                plsc.addupdate(tok_v.at[r, sl], pos_v[r, sl])
            return carry

        if ch == 0:
            @pl.when(wid == 0)
            def _():
                for k in range(VPR):
                    sl = pl.ds(k * LANES, LANES)
                    stage_v[0, sl] = tok_v[0, sl] + pos_v[0, sl]

                def add_shift(r, carry):
                    for k in range(VPR):
                        sl = pl.ds(k * LANES, LANES)
                        stage_v[NSPECIAL + r, sl] = tok_v[r, sl] + pos_v[r, sl]
                    return carry
                lax.fori_loop(1, CHUNK, add_shift, 0)
                pltpu.sync_copy(stage_v, out_hbm.at[pl.ds(0, HEAD)])

            @pl.when(wid != 0)
            def _():
                lax.fori_loop(0, CHUNK, add_row, 0)
                pltpu.sync_copy(tok_v, out_hbm.at[pl.ds(row0 + NSPECIAL, CHUNK)])
        else:
            lax.fori_loop(0, CHUNK, add_row, 0)
            pltpu.sync_copy(tok_v, out_hbm.at[pl.ds(row0 + NSPECIAL, CHUNK)])


_sc_kernel = functools.partial(
    pl.kernel,
    out_type=jax.ShapeDtypeStruct((LROWS, DIM), jnp.float32),
    mesh=plsc.VectorSubcoreMesh(core_axis_name="c", subcore_axis_name="s"),
    scratch_types=[
        pltpu.VMEM((CHUNK,), jnp.int32),
        pltpu.VMEM((CHUNK, DIM), jnp.float32),
        pltpu.VMEM((CHUNK, DIM), jnp.float32),
        pltpu.VMEM((HEAD, DIM), jnp.float32),
        pltpu.SemaphoreType.DMA,
        pltpu.SemaphoreType.DMA,
    ],
)(_sc_body)


def kernel(input_ids, token_table, pos_table, special_token_embedding):
    ids = input_ids.reshape(LROWS)
    spec = special_token_embedding.reshape(NSPECIAL, DIM)
    out = _sc_kernel(ids, token_table, pos_table, spec)
    return out.reshape(1, LROWS, DIM)


# speedup vs baseline: 3.5343x; 1.2007x over previous
import functools

import jax
import jax.numpy as jnp
from jax import lax
from jax.experimental import pallas as pl
from jax.experimental.pallas import tpu as pltpu
from jax.experimental.pallas import tpu_sc as plsc

VOCAB = 49408
MAXPOS = 8192
DIM = 1024
NSPECIAL = 16
LROWS = MAXPOS + NSPECIAL

NC = 2
NS = 16
LANES = 16
NW = NC * NS
RW = MAXPOS // NW
CHUNK = 16
NCHUNK = RW // CHUNK
VPR = DIM // LANES
HEAD = NSPECIAL + CHUNK


def _sc_body(ids_hbm, tok_hbm, pos_hbm, spec_hbm, out_hbm,
             idx_all, tok_v, pos_v, stage_v,
             gsem0, gsem1, psem0, psem1, wsem0, wsem1):
    gsem = (gsem0, gsem1)
    psem = (psem0, psem1)
    wsem = (wsem0, wsem1)
    wid = lax.axis_index("s") * NC + lax.axis_index("c")
    base = wid * RW

    pltpu.sync_copy(ids_hbm.at[pl.ds(NSPECIAL + base, RW)], idx_all)

    @pl.when(wid == 0)
    def _():
        pltpu.sync_copy(spec_hbm, stage_v.at[pl.ds(CHUNK, NSPECIAL)])

        def cp_row(r, carry):
            for k in range(VPR):
                sl = pl.ds(k * LANES, LANES)
                stage_v[1 + r, sl] = stage_v[CHUNK + r, sl]
            return carry
        lax.fori_loop(0, NSPECIAL, cp_row, 0)

    def issue(ch, b):
        row0 = base + ch * CHUNK
        g = pltpu.async_copy(tok_hbm.at[idx_all.at[pl.ds(ch * CHUNK, CHUNK)]],
                             tok_v.at[b], gsem[b])
        p = pltpu.async_copy(pos_hbm.at[pl.ds(row0, CHUNK)],
                             pos_v.at[b], psem[b])
        return g, p

    inflight = [None, None]
    writes = [None, None]
    inflight[0] = issue(0, 0)

    for ch in range(NCHUNK):
        b = ch & 1
        if ch + 1 < NCHUNK:
            if writes[1 - b] is not None:
                writes[1 - b].wait()
                writes[1 - b] = None
            inflight[1 - b] = issue(ch + 1, 1 - b)
        g, p = inflight[b]
        g.wait()
        p.wait()

        def add_row(r, carry, _b=b):
            for k in range(VPR):
                sl = pl.ds(k * LANES, LANES)
                plsc.addupdate(tok_v.at[_b, r, sl], pos_v[_b, r, sl])
            return carry

        if ch == 0:
            @pl.when(wid == 0)
            def _():
                for k in range(VPR):
                    sl = pl.ds(k * LANES, LANES)
                    stage_v[0, sl] = tok_v[b, 0, sl] + pos_v[b, 0, sl]

                def add_shift(r, carry):
                    for k in range(VPR):
                        sl = pl.ds(k * LANES, LANES)
                        stage_v[NSPECIAL + r, sl] = (
                            tok_v[b, r, sl] + pos_v[b, r, sl])
                    return carry
                lax.fori_loop(1, CHUNK, add_shift, 0)
                pltpu.sync_copy(stage_v, out_hbm.at[pl.ds(0, HEAD)])

            @pl.when(wid != 0)
            def _():
                lax.fori_loop(0, CHUNK, add_row, 0)
                pltpu.sync_copy(tok_v.at[b],
                                out_hbm.at[pl.ds(base + NSPECIAL, CHUNK)])
        else:
            lax.fori_loop(0, CHUNK, add_row, 0)
            row0 = base + ch * CHUNK
            writes[b] = pltpu.async_copy(
                tok_v.at[b], out_hbm.at[pl.ds(row0 + NSPECIAL, CHUNK)],
                wsem[b])

    for w in writes:
        if w is not None:
            w.wait()


_sc_kernel = functools.partial(
    pl.kernel,
    out_type=jax.ShapeDtypeStruct((LROWS, DIM), jnp.float32),
    mesh=plsc.VectorSubcoreMesh(core_axis_name="c", subcore_axis_name="s"),
    scratch_types=[
        pltpu.VMEM((RW,), jnp.int32),
        pltpu.VMEM((2, CHUNK, DIM), jnp.float32),
        pltpu.VMEM((2, CHUNK, DIM), jnp.float32),
        pltpu.VMEM((HEAD, DIM), jnp.float32),
        pltpu.SemaphoreType.DMA,
        pltpu.SemaphoreType.DMA,
        pltpu.SemaphoreType.DMA,
        pltpu.SemaphoreType.DMA,
        pltpu.SemaphoreType.DMA,
        pltpu.SemaphoreType.DMA,
    ],
)(_sc_body)


def kernel(input_ids, token_table, pos_table, special_token_embedding):
    ids = input_ids.reshape(LROWS)
    spec = special_token_embedding.reshape(NSPECIAL, DIM)
    out = _sc_kernel(ids, token_table, pos_table, spec)
    return out.reshape(1, LROWS, DIM)
